# baseline (speedup 1.0000x reference)
import functools

import jax
import jax.numpy as jnp
from jax import lax
from jax.experimental import pallas as pl
from jax.experimental.pallas import tpu as pltpu
from jax.experimental.pallas import tpu_sc as plsc

_NZ = 2048
_RGRID0 = -4.0
_ZGRID0 = -4.0
_H = 0.00390625
_INV_H = 256.0
_SCALE = 65536.0
_IMAX = 2046.0

_NC = 2
_NS = 16
_NW = _NC * _NS
_LANES = 16

_CHUNK = 2048
_SLICE = 128
_NSLICE = _CHUNK // _SLICE
_MASKHI = -65536


def _hi_f32(w):
    return lax.bitcast_convert_type(w & _MASKHI, jnp.float32)


def _lo_f32(w):
    return lax.bitcast_convert_type(w << 16, jnp.float32)


def _corner_i(v, grid0):
    scaled = (v - grid0) * _INV_H
    return jnp.minimum(jnp.maximum(scaled, 0.0), _IMAX).astype(jnp.int32)


@jax.jit
def _run(r, z, timetable):
    n = r.shape[0]
    nchunk = n // _NW // _CHUNK
    mesh = plsc.VectorSubcoreMesh(core_axis_name="c", subcore_axis_name="s")

    @functools.partial(
        pl.kernel,
        out_type=jax.ShapeDtypeStruct((n,), jnp.float32),
        mesh=mesh,
        scratch_types=[
            pltpu.VMEM((2, _CHUNK), jnp.float32),
            pltpu.VMEM((2, _CHUNK), jnp.float32),
            pltpu.VMEM((2, 2, _NSLICE, _SLICE), jnp.int32),
            pltpu.VMEM((2, 2 * _CHUNK), jnp.int32),
            pltpu.VMEM((2, _CHUNK), jnp.float32),
            pltpu.SemaphoreType.DMA,
            pltpu.SemaphoreType.DMA,
            pltpu.SemaphoreType.DMA,
            pltpu.SemaphoreType.DMA,
            pltpu.SemaphoreType.DMA,
            pltpu.SemaphoreType.DMA,
        ],
    )
    def body(r_hbm, z_hbm, tt_hbm, out_hbm, r_v, z_v, idx_v, q_v, o_v,
             isem0, isem1, gsem0, gsem1, osem0, osem1):
        qpw = n // _NW
        wid = lax.axis_index("s") * _NC + lax.axis_index("c")
        base = wid * qpw

        bufs = (
            (r_v.at[0], z_v.at[0], idx_v.at[0], q_v.at[0], o_v.at[0],
             isem0, gsem0, osem0),
            (r_v.at[1], z_v.at[1], idx_v.at[1], q_v.at[1], o_v.at[1],
             isem1, gsem1, osem1),
        )

        def stage_in(c, buf):
            rb, zb, _, _, _, isem, _, _ = buf
            off = base + c * _CHUNK
            pltpu.async_copy(r_hbm.at[pl.ds(off, _CHUNK)], rb, isem)
            pltpu.async_copy(z_hbm.at[pl.ds(off, _CHUNK)], zb, isem)

        def stage_idx_fire(c, buf):
            rb, zb, ib, qb, _, isem, gsem, _ = buf
            off = base + c * _CHUNK
            pltpu.make_async_copy(r_hbm.at[pl.ds(off, _CHUNK)], rb, isem).wait()
            pltpu.make_async_copy(z_hbm.at[pl.ds(off, _CHUNK)], zb, isem).wait()

            @plsc.parallel_loop(0, _NSLICE)
            def _indices(j):
                for t in range(_SLICE // _LANES):
                    i = j * _SLICE + t * _LANES
                    ir = _corner_i(rb[pl.ds(i, _LANES)], _RGRID0)
                    iz = _corner_i(zb[pl.ds(i, _LANES)], _ZGRID0)
                    i00 = ir * _NZ + iz
                    s = pl.ds(t * _LANES, _LANES)
                    ib[0, j, s] = i00
                    ib[1, j, s] = i00 + _NZ

            @pl.loop(0, _NSLICE)
            def _fire(j):
                for k in range(2):
                    pltpu.async_copy(
                        tt_hbm.at[ib.at[k, j]],
                        qb.at[pl.ds(k * _CHUNK + j * _SLICE, _SLICE)], gsem)

        def stage_finish(c, buf):
            rb, zb, ib, qb, ob, _, gsem, osem = buf
            off = base + c * _CHUNK

            pltpu.make_async_copy(tt_hbm.at[pl.ds(0, 2 * _CHUNK)], qb, gsem).wait()

            @pl.when(c >= 2)
            def _wait_prev_out():
                pltpu.make_async_copy(
                    ob, out_hbm.at[pl.ds(off, _CHUNK)], osem).wait()

            @plsc.parallel_loop(0, _NSLICE)
            def _combine(j):
                for t in range(_SLICE // _LANES):
                    i = j * _SLICE + t * _LANES
                    s = pl.ds(i, _LANES)
                    ts = pl.ds(t * _LANES, _LANES)
                    rv = rb[s]
                    zv = zb[s]
                    i00 = ib[0, j, ts]
                    ir = i00 >> 11
                    iz = i00 & (_NZ - 1)
                    irf = ir.astype(jnp.float32)
                    izf = iz.astype(jnp.float32)
                    x1 = irf * _H + _RGRID0
                    x2 = x1 + _H
                    y1 = izf * _H + _ZGRID0
                    y2 = y1 + _H
                    wx2 = x2 - rv
                    wx1 = rv - x1
                    wy2 = y2 - zv
                    wy1 = zv - y1
                    wa = qb[pl.ds(i, _LANES)]
                    wb = qb[pl.ds(_CHUNK + i, _LANES)]
                    q11 = _hi_f32(wa)
                    q12 = _lo_f32(wa)
                    q21 = _hi_f32(wb)
                    q22 = _lo_f32(wb)
                    acc = ((q11 * wx2 + q21 * wx1) * wy2
                           + (q12 * wx2 + q22 * wx1) * wy1)
                    ob[s] = _SCALE * acc

            pltpu.async_copy(ob, out_hbm.at[pl.ds(off, _CHUNK)], osem)

        stage_in(0, bufs[0])
        stage_in(1, bufs[1])
        stage_idx_fire(0, bufs[0])

        @pl.loop(0, nchunk // 2 - 1)
        def _steady(i):
            c0 = 2 * i
            plsc.subcore_barrier()
            stage_idx_fire(c0 + 1, bufs[1])
            stage_finish(c0, bufs[0])
            stage_in(c0 + 2, bufs[0])
            stage_idx_fire(c0 + 2, bufs[0])
            stage_finish(c0 + 1, bufs[1])
            stage_in(c0 + 3, bufs[1])

        stage_idx_fire(nchunk - 1, bufs[1])
        stage_finish(nchunk - 2, bufs[0])
        stage_finish(nchunk - 1, bufs[1])

        pltpu.make_async_copy(
            o_v.at[0], out_hbm.at[pl.ds(base + (nchunk - 2) * _CHUNK, _CHUNK)],
            osem0).wait()
        pltpu.make_async_copy(
            o_v.at[1], out_hbm.at[pl.ds(base + (nchunk - 1) * _CHUNK, _CHUNK)],
            osem1).wait()

    return body(r, z, timetable)


def kernel(r, z, timetable):
    t16 = lax.convert_element_type(timetable, jnp.bfloat16)
    hi = lax.bitcast_convert_type(t16, jnp.uint16).astype(jnp.int32)
    lo = lax.bitcast_convert_type(jnp.roll(t16, -1), jnp.uint16).astype(jnp.int32)
    packed = (hi << 16) | lo
    return _run(r, z, packed)


# speedup vs baseline: 1.0020x; 1.0020x over previous
import functools

import jax
import jax.numpy as jnp
from jax import lax
from jax.experimental import pallas as pl
from jax.experimental.pallas import tpu as pltpu
from jax.experimental.pallas import tpu_sc as plsc

_NZ = 2048
_RGRID0 = -4.0
_ZGRID0 = -4.0
_H = 0.00390625
_INV_H = 256.0
_SCALE = 65536.0
_IMAX = 2046.0

_NC = 2
_NS = 16
_NW = _NC * _NS
_LANES = 16

_CHUNK = 2048
_SLICE = 128
_NSLICE = _CHUNK // _SLICE
_MASKHI = -65536


def _hi_f32(w):
    return lax.bitcast_convert_type(w & _MASKHI, jnp.float32)


def _lo_f32(w):
    return lax.bitcast_convert_type(w << 16, jnp.float32)


def _corner_i(v, grid0):
    scaled = (v - grid0) * _INV_H
    return jnp.minimum(jnp.maximum(scaled, 0.0), _IMAX).astype(jnp.int32)


@jax.jit
def _run(r, z, timetable):
    n = r.shape[0]
    nchunk = n // _NW // _CHUNK
    mesh = plsc.VectorSubcoreMesh(core_axis_name="c", subcore_axis_name="s")

    @functools.partial(
        pl.kernel,
        out_type=jax.ShapeDtypeStruct((n,), jnp.float32),
        mesh=mesh,
        scratch_types=[
            pltpu.VMEM((2, _CHUNK), jnp.float32),
            pltpu.VMEM((2, _CHUNK), jnp.float32),
            pltpu.VMEM((2, 2, _NSLICE, _SLICE), jnp.int32),
            pltpu.VMEM((2, 2 * _CHUNK), jnp.int32),
            pltpu.VMEM((2, _CHUNK), jnp.float32),
            pltpu.SemaphoreType.DMA,
            pltpu.SemaphoreType.DMA,
            pltpu.SemaphoreType.DMA,
            pltpu.SemaphoreType.DMA,
            pltpu.SemaphoreType.DMA,
            pltpu.SemaphoreType.DMA,
        ],
    )
    def body(r_hbm, z_hbm, tt_hbm, out_hbm, r_v, z_v, idx_v, q_v, o_v,
             isem0, isem1, gsem0, gsem1, osem0, osem1):
        qpw = n // _NW
        wid = lax.axis_index("s") * _NC + lax.axis_index("c")
        base = wid * qpw

        bufs = (
            (r_v.at[0], z_v.at[0], idx_v.at[0], q_v.at[0], o_v.at[0],
             isem0, gsem0, osem0),
            (r_v.at[1], z_v.at[1], idx_v.at[1], q_v.at[1], o_v.at[1],
             isem1, gsem1, osem1),
        )

        def stage_in(c, buf):
            rb, zb, _, _, _, isem, _, _ = buf
            off = base + c * _CHUNK
            pltpu.async_copy(r_hbm.at[pl.ds(off, _CHUNK)], rb, isem)
            pltpu.async_copy(z_hbm.at[pl.ds(off, _CHUNK)], zb, isem)

        def stage_idx_fire(c, buf):
            rb, zb, ib, qb, _, isem, gsem, _ = buf
            off = base + c * _CHUNK
            pltpu.make_async_copy(r_hbm.at[pl.ds(off, _CHUNK)], rb, isem).wait()
            pltpu.make_async_copy(z_hbm.at[pl.ds(off, _CHUNK)], zb, isem).wait()

            @pl.loop(0, _NSLICE)
            def _indices(j):
                for t in range(_SLICE // _LANES):
                    i = j * _SLICE + t * _LANES
                    ir = _corner_i(rb[pl.ds(i, _LANES)], _RGRID0)
                    iz = _corner_i(zb[pl.ds(i, _LANES)], _ZGRID0)
                    i00 = ir * _NZ + iz
                    s = pl.ds(t * _LANES, _LANES)
                    ib[0, j, s] = i00
                    ib[1, j, s] = i00 + _NZ

            @pl.loop(0, _NSLICE)
            def _fire(j):
                for k in range(2):
                    pltpu.async_copy(
                        tt_hbm.at[ib.at[k, j]],
                        qb.at[pl.ds(k * _CHUNK + j * _SLICE, _SLICE)], gsem)

        def stage_finish(c, buf):
            rb, zb, ib, qb, ob, _, gsem, osem = buf
            off = base + c * _CHUNK

            pltpu.make_async_copy(tt_hbm.at[pl.ds(0, 2 * _CHUNK)], qb, gsem).wait()

            @pl.when(c >= 2)
            def _wait_prev_out():
                pltpu.make_async_copy(
                    ob, out_hbm.at[pl.ds(off, _CHUNK)], osem).wait()

            @pl.loop(0, _NSLICE)
            def _combine(j):
                for t in range(_SLICE // _LANES):
                    i = j * _SLICE + t * _LANES
                    s = pl.ds(i, _LANES)
                    ts = pl.ds(t * _LANES, _LANES)
                    rv = rb[s]
                    zv = zb[s]
                    i00 = ib[0, j, ts]
                    ir = i00 >> 11
                    iz = i00 & (_NZ - 1)
                    irf = ir.astype(jnp.float32)
                    izf = iz.astype(jnp.float32)
                    x1 = irf * _H + _RGRID0
                    x2 = x1 + _H
                    y1 = izf * _H + _ZGRID0
                    y2 = y1 + _H
                    wx2 = x2 - rv
                    wx1 = rv - x1
                    wy2 = y2 - zv
                    wy1 = zv - y1
                    wa = qb[pl.ds(i, _LANES)]
                    wb = qb[pl.ds(_CHUNK + i, _LANES)]
                    q11 = _hi_f32(wa)
                    q12 = _lo_f32(wa)
                    q21 = _hi_f32(wb)
                    q22 = _lo_f32(wb)
                    acc = ((q11 * wx2 + q21 * wx1) * wy2
                           + (q12 * wx2 + q22 * wx1) * wy1)
                    ob[s] = _SCALE * acc

            pltpu.async_copy(ob, out_hbm.at[pl.ds(off, _CHUNK)], osem)

        stage_in(0, bufs[0])
        stage_in(1, bufs[1])
        stage_idx_fire(0, bufs[0])

        @pl.loop(0, nchunk // 2 - 1)
        def _steady(i):
            c0 = 2 * i
            plsc.subcore_barrier()
            stage_idx_fire(c0 + 1, bufs[1])
            stage_finish(c0, bufs[0])
            stage_in(c0 + 2, bufs[0])
            stage_idx_fire(c0 + 2, bufs[0])
            stage_finish(c0 + 1, bufs[1])
            stage_in(c0 + 3, bufs[1])

        stage_idx_fire(nchunk - 1, bufs[1])
        stage_finish(nchunk - 2, bufs[0])
        stage_finish(nchunk - 1, bufs[1])

        pltpu.make_async_copy(
            o_v.at[0], out_hbm.at[pl.ds(base + (nchunk - 2) * _CHUNK, _CHUNK)],
            osem0).wait()
        pltpu.make_async_copy(
            o_v.at[1], out_hbm.at[pl.ds(base + (nchunk - 1) * _CHUNK, _CHUNK)],
            osem1).wait()

    return body(r, z, timetable)


def kernel(r, z, timetable):
    t16 = lax.convert_element_type(timetable, jnp.bfloat16)
    hi = lax.bitcast_convert_type(t16, jnp.uint16).astype(jnp.int32)
    lo = lax.bitcast_convert_type(jnp.roll(t16, -1), jnp.uint16).astype(jnp.int32)
    packed = (hi << 16) | lo
    return _run(r, z, packed)
